# Initial kernel scaffold; baseline (speedup 1.0000x reference)
import jax
import jax.numpy as jnp
from jax.experimental import pallas as pl


def kernel(x, edge_index, W1l, b1, W1r, W2l, b2, W2r, W3l, b3, W3r, a, Wp, bp):
    raise NotImplementedError("write your pallas kernel here")


# speedup vs baseline: 3.9560x; 3.9560x over previous
import functools

import jax
import jax.numpy as jnp
from jax import lax
from jax.experimental import pallas as pl
from jax.experimental.pallas import tpu as pltpu
from jax.experimental.pallas import tpu_sc as plsc

N = 10000
E = 320000
NP = 10240
D = 128
BLK = 2048
NT = 32
DPT = D // NT
EC = 8000
EPT = E // NT


def _feat_segsum_body(yt_hbm, src_hbm, dst_hbm, out_hbm, tab, acc, sbuf, dbuf):
    wid = lax.axis_index("s") * 2 + lax.axis_index("c")
    r0 = wid * DPT
    pltpu.sync_copy(yt_hbm.at[pl.ds(r0, DPT)], tab)

    zero = jnp.zeros((16,), jnp.float32)

    def zbody(i, carry):
        for d in range(DPT):
            acc[d, pl.ds(i * 16, 16)] = zero
        return carry

    lax.fori_loop(0, NP // 16, zbody, 0)

    dvecs = [jnp.full((16,), d, jnp.int32) for d in range(DPT)]

    def cbody(ci, carry):
        pltpu.sync_copy(src_hbm.at[pl.ds(ci * EC, EC)], sbuf)
        pltpu.sync_copy(dst_hbm.at[pl.ds(ci * EC, EC)], dbuf)

        def gbody(g, c2):
            sv = sbuf[pl.ds(g * 16, 16)]
            dv = dbuf[pl.ds(g * 16, 16)]
            for d in range(DPT):
                vals = plsc.load_gather(tab, [dvecs[d], sv])
                plsc.addupdate_scatter(acc, [dvecs[d], dv], vals)
            return c2

        lax.fori_loop(0, EC // 16, gbody, 0)
        return carry

    lax.fori_loop(0, E // EC, cbody, 0)
    pltpu.sync_copy(acc, out_hbm.at[pl.ds(r0, DPT)])


def _scalar_segsum_body(tab_hbm, src_hbm, dst_hbm, out_hbm, tab, acc, sbuf, dbuf):
    wid = lax.axis_index("s") * 2 + lax.axis_index("c")
    base = wid * EPT
    pltpu.sync_copy(tab_hbm, tab)
    pltpu.sync_copy(src_hbm.at[pl.ds(base, EPT)], sbuf)
    pltpu.sync_copy(dst_hbm.at[pl.ds(base, EPT)], dbuf)

    zero = jnp.zeros((16,), jnp.float32)

    def zbody(i, carry):
        acc[pl.ds(i * 16, 16)] = zero
        return carry

    lax.fori_loop(0, NP // 16, zbody, 0)

    def gbody(g, carry):
        sv = sbuf[pl.ds(g * 16, 16)]
        dv = dbuf[pl.ds(g * 16, 16)]
        vals = plsc.load_gather(tab, [sv])
        plsc.addupdate_scatter(acc, [dv], vals)
        return carry

    lax.fori_loop(0, EPT // 16, gbody, 0)
    pltpu.sync_copy(acc, out_hbm.at[wid])


@functools.cache
def _sc_kernels():
    mesh = plsc.VectorSubcoreMesh(core_axis_name="c", subcore_axis_name="s")
    params = pltpu.CompilerParams(needs_layout_passes=False)
    feat = pl.kernel(
        _feat_segsum_body,
        mesh=mesh,
        compiler_params=params,
        out_type=jax.ShapeDtypeStruct((D, NP), jnp.float32),
        scratch_types=[
            pltpu.VMEM((DPT, NP), jnp.float32),
            pltpu.VMEM((DPT, NP), jnp.float32),
            pltpu.VMEM((EC,), jnp.int32),
            pltpu.VMEM((EC,), jnp.int32),
        ],
    )
    scalar = pl.kernel(
        _scalar_segsum_body,
        mesh=mesh,
        compiler_params=params,
        out_type=jax.ShapeDtypeStruct((NT, NP), jnp.float32),
        scratch_types=[
            pltpu.VMEM((NP,), jnp.float32),
            pltpu.VMEM((NP,), jnp.float32),
            pltpu.VMEM((EPT,), jnp.int32),
            pltpu.VMEM((EPT,), jnp.int32),
        ],
    )
    return feat, scalar


_DN_T = (((1,), (1,)), ((), ()))
_DN = (((1,), (0,)), ((), ()))


def _l1_body(x_ref, wl_ref, wr_ref, b_ref, y_ref, z_ref):
    xb = x_ref[...]
    y_ref[...] = lax.dot_general(wl_ref[...], xb, _DN_T,
                                 preferred_element_type=jnp.float32)
    z_ref[...] = lax.dot_general(wr_ref[...], xb, _DN_T,
                                 preferred_element_type=jnp.float32) + b_ref[...]


def _mid_body(agg_ref, z_ref, dinv_ref, a_ref, wl_ref, wr_ref, b_ref,
              y_ref, zo_ref):
    pre = agg_ref[...] * dinv_ref[...] + z_ref[...]
    h = jnp.where(pre >= 0, pre, a_ref[...] * pre)
    y_ref[...] = lax.dot_general(wl_ref[...], h, _DN,
                                 preferred_element_type=jnp.float32)
    zo_ref[...] = lax.dot_general(wr_ref[...], h, _DN,
                                  preferred_element_type=jnp.float32) + b_ref[...]


def _head_body(agg_ref, z_ref, dinv_ref, a_ref, uv_ref, st_ref):
    pre = agg_ref[...] * dinv_ref[...] + z_ref[...]
    h = jnp.where(pre >= 0, pre, a_ref[...] * pre)
    st_ref[...] = lax.dot_general(uv_ref[...], h, _DN,
                                  preferred_element_type=jnp.float32)


def _dinv_body(cnt_ref, dinv_ref):
    s = jnp.sum(cnt_ref[...], axis=0, keepdims=True)
    dinv_ref[...] = 1.0 / jnp.maximum(s, 1.0)


def _final_body(part_ref, st_ref, dinv_ref, c_ref, out_ref):
    s = jnp.sum(part_ref[...], axis=0, keepdims=True)
    out_ref[...] = s * dinv_ref[...] + st_ref[1:2, :] + c_ref[...]


def _full(shape):
    return pl.BlockSpec(shape, lambda j: (0,) * len(shape))


def _tc_l1(x, wl, wr, b):
    return pl.pallas_call(
        _l1_body,
        grid=(NP // BLK,),
        in_specs=[pl.BlockSpec((BLK, D), lambda j: (j, 0)),
                  _full((D, D)), _full((D, D)), _full((D, 1))],
        out_specs=[pl.BlockSpec((D, BLK), lambda j: (0, j)),
                   pl.BlockSpec((D, BLK), lambda j: (0, j))],
        out_shape=[jax.ShapeDtypeStruct((D, NP), jnp.float32)] * 2,
    )(x, wl, wr, b)


def _tc_mid(agg, z, dinv, a, wl, wr, b):
    return pl.pallas_call(
        _mid_body,
        grid=(NP // BLK,),
        in_specs=[pl.BlockSpec((D, BLK), lambda j: (0, j)),
                  pl.BlockSpec((D, BLK), lambda j: (0, j)),
                  pl.BlockSpec((1, BLK), lambda j: (0, j)),
                  _full((1, 1)), _full((D, D)), _full((D, D)), _full((D, 1))],
        out_specs=[pl.BlockSpec((D, BLK), lambda j: (0, j)),
                   pl.BlockSpec((D, BLK), lambda j: (0, j))],
        out_shape=[jax.ShapeDtypeStruct((D, NP), jnp.float32)] * 2,
    )(agg, z, dinv, a, wl, wr, b)


def _tc_head(agg, z, dinv, a, uv):
    return pl.pallas_call(
        _head_body,
        grid=(NP // BLK,),
        in_specs=[pl.BlockSpec((D, BLK), lambda j: (0, j)),
                  pl.BlockSpec((D, BLK), lambda j: (0, j)),
                  pl.BlockSpec((1, BLK), lambda j: (0, j)),
                  _full((1, 1)), _full((2, D))],
        out_specs=pl.BlockSpec((2, BLK), lambda j: (0, j)),
        out_shape=jax.ShapeDtypeStruct((2, NP), jnp.float32),
    )(agg, z, dinv, a, uv)


def _tc_dinv(cnt):
    return pl.pallas_call(
        _dinv_body,
        grid=(NP // BLK,),
        in_specs=[pl.BlockSpec((NT, BLK), lambda j: (0, j))],
        out_specs=pl.BlockSpec((1, BLK), lambda j: (0, j)),
        out_shape=jax.ShapeDtypeStruct((1, NP), jnp.float32),
    )(cnt)


def _tc_final(part, st, dinv, c):
    return pl.pallas_call(
        _final_body,
        grid=(NP // BLK,),
        in_specs=[pl.BlockSpec((NT, BLK), lambda j: (0, j)),
                  pl.BlockSpec((2, BLK), lambda j: (0, j)),
                  pl.BlockSpec((1, BLK), lambda j: (0, j)),
                  _full((1, 1))],
        out_specs=pl.BlockSpec((1, BLK), lambda j: (0, j)),
        out_shape=jax.ShapeDtypeStruct((1, NP), jnp.float32),
    )(part, st, dinv, c)


def kernel(x, edge_index, W1l, b1, W1r, W2l, b2, W2r, W3l, b3, W3r, a, Wp, bp):
    src = edge_index[0]
    dst = edge_index[1]
    xp = jnp.pad(x, ((0, NP - N), (0, 0)))
    ones = jnp.ones((NP,), jnp.float32)
    a2 = jnp.reshape(a, (1, 1))
    b1c = jnp.reshape(b1, (D, 1))
    b2c = jnp.reshape(b2, (D, 1))
    uv = jnp.concatenate([Wp @ W3l, Wp @ W3r], axis=0)
    c = jnp.reshape(Wp @ b3 + bp, (1, 1))

    feat_segsum, scalar_segsum = _sc_kernels()
    cnt = scalar_segsum(ones, src, dst)
    dinv = _tc_dinv(cnt)
    y1, z1 = _tc_l1(xp, W1l, W1r, b1c)
    agg1 = feat_segsum(y1, src, dst)
    y2, z2 = _tc_mid(agg1, z1, dinv, a2, W2l, W2r, b2c)
    agg2 = feat_segsum(y2, src, dst)
    st = _tc_head(agg2, z2, dinv, a2, uv)
    spart = scalar_segsum(st[0], src, dst)
    out = _tc_final(spart, st, dinv, c)
    return out[0, :N]


# speedup vs baseline: 11.6599x; 2.9474x over previous
import functools

import jax
import jax.numpy as jnp
from jax import lax
from jax.experimental import pallas as pl
from jax.experimental.pallas import tpu as pltpu
from jax.experimental.pallas import tpu_sc as plsc

N = 10000
E = 320000
NP = 10240
D = 128
BLK = 2048
NT = 32
DPT = D // NT
EC = 8000
EPT = E // NT


def _feat_segsum_body(yt_hbm, src_hbm, dst_hbm, out_hbm, tab, acc,
                      sbuf0, dbuf0, sbuf1, dbuf1,
                      sem_t, sem_s0, sem_d0, sem_s1, sem_d1):
    wid = lax.axis_index("s") * 2 + lax.axis_index("c")
    r0 = wid * DPT
    tab_cp = pltpu.async_copy(yt_hbm.at[pl.ds(r0, DPT)], tab, sem_t)
    pltpu.async_copy(src_hbm.at[pl.ds(0, EC)], sbuf0, sem_s0)
    pltpu.async_copy(dst_hbm.at[pl.ds(0, EC)], dbuf0, sem_d0)

    zero = jnp.zeros((16,), jnp.float32)

    @plsc.parallel_loop(0, NP // 16, unroll=8)
    def _zero(i):
        for d in range(DPT):
            acc[d, pl.ds(i * 16, 16)] = zero

    tab_cp.wait()

    dvecs = [jnp.full((16,), d, jnp.int32) for d in range(DPT)]

    def _process(sb, db):
        @plsc.parallel_loop(0, EC // 16, unroll=4)
        def _groups(g):
            sv = sb[pl.ds(g * 16, 16)]
            dv = db[pl.ds(g * 16, 16)]
            for d in range(DPT):
                vals = plsc.load_gather(tab, [dvecs[d], sv])
                plsc.addupdate_scatter(acc, [dvecs[d], dv], vals)

    def cbody(i, carry):
        c0 = 2 * i
        c1 = 2 * i + 1
        pltpu.async_copy(src_hbm.at[pl.ds(c1 * EC, EC)], sbuf1, sem_s1)
        pltpu.async_copy(dst_hbm.at[pl.ds(c1 * EC, EC)], dbuf1, sem_d1)
        pltpu.make_async_copy(src_hbm.at[pl.ds(c0 * EC, EC)], sbuf0, sem_s0).wait()
        pltpu.make_async_copy(dst_hbm.at[pl.ds(c0 * EC, EC)], dbuf0, sem_d0).wait()
        _process(sbuf0, dbuf0)
        nxt = c0 + 2

        @pl.when(nxt < E // EC)
        def _():
            pltpu.async_copy(src_hbm.at[pl.ds(nxt * EC, EC)], sbuf0, sem_s0)
            pltpu.async_copy(dst_hbm.at[pl.ds(nxt * EC, EC)], dbuf0, sem_d0)

        pltpu.make_async_copy(src_hbm.at[pl.ds(c1 * EC, EC)], sbuf1, sem_s1).wait()
        pltpu.make_async_copy(dst_hbm.at[pl.ds(c1 * EC, EC)], dbuf1, sem_d1).wait()
        _process(sbuf1, dbuf1)
        return carry

    lax.fori_loop(0, E // EC // 2, cbody, 0)
    pltpu.sync_copy(acc, out_hbm.at[pl.ds(r0, DPT)])


def _scalar_segsum_body(tab_hbm, src_hbm, dst_hbm, out_hbm, tab, acc, sbuf, dbuf):
    wid = lax.axis_index("s") * 2 + lax.axis_index("c")
    base = wid * EPT
    pltpu.sync_copy(tab_hbm, tab)
    pltpu.sync_copy(src_hbm.at[pl.ds(base, EPT)], sbuf)
    pltpu.sync_copy(dst_hbm.at[pl.ds(base, EPT)], dbuf)

    zero = jnp.zeros((16,), jnp.float32)

    @plsc.parallel_loop(0, NP // 16, unroll=8)
    def _zero(i):
        acc[pl.ds(i * 16, 16)] = zero

    @plsc.parallel_loop(0, EPT // 16, unroll=8)
    def _groups(g):
        sv = sbuf[pl.ds(g * 16, 16)]
        dv = dbuf[pl.ds(g * 16, 16)]
        vals = plsc.load_gather(tab, [sv])
        plsc.addupdate_scatter(acc, [dv], vals)

    pltpu.sync_copy(acc, out_hbm.at[wid])


@functools.cache
def _sc_kernels():
    mesh = plsc.VectorSubcoreMesh(core_axis_name="c", subcore_axis_name="s")
    params = pltpu.CompilerParams(needs_layout_passes=False)
    feat = pl.kernel(
        _feat_segsum_body,
        mesh=mesh,
        compiler_params=params,
        out_type=jax.ShapeDtypeStruct((D, NP), jnp.float32),
        scratch_types=[
            pltpu.VMEM((DPT, NP), jnp.float32),
            pltpu.VMEM((DPT, NP), jnp.float32),
            pltpu.VMEM((EC,), jnp.int32),
            pltpu.VMEM((EC,), jnp.int32),
            pltpu.VMEM((EC,), jnp.int32),
            pltpu.VMEM((EC,), jnp.int32),
            pltpu.SemaphoreType.DMA,
            pltpu.SemaphoreType.DMA,
            pltpu.SemaphoreType.DMA,
            pltpu.SemaphoreType.DMA,
            pltpu.SemaphoreType.DMA,
        ],
    )
    scalar = pl.kernel(
        _scalar_segsum_body,
        mesh=mesh,
        compiler_params=params,
        out_type=jax.ShapeDtypeStruct((NT, NP), jnp.float32),
        scratch_types=[
            pltpu.VMEM((NP,), jnp.float32),
            pltpu.VMEM((NP,), jnp.float32),
            pltpu.VMEM((EPT,), jnp.int32),
            pltpu.VMEM((EPT,), jnp.int32),
        ],
    )
    return feat, scalar


_DN_T = (((1,), (1,)), ((), ()))
_DN = (((1,), (0,)), ((), ()))


def _l1_body(x_ref, wl_ref, wr_ref, b_ref, y_ref, z_ref):
    xb = x_ref[...]
    y_ref[...] = lax.dot_general(wl_ref[...], xb, _DN_T,
                                 preferred_element_type=jnp.float32)
    z_ref[...] = lax.dot_general(wr_ref[...], xb, _DN_T,
                                 preferred_element_type=jnp.float32) + b_ref[...]


def _mid_body(agg_ref, z_ref, dinv_ref, a_ref, wl_ref, wr_ref, b_ref,
              y_ref, zo_ref):
    pre = agg_ref[...] * dinv_ref[...] + z_ref[...]
    h = jnp.where(pre >= 0, pre, a_ref[...] * pre)
    y_ref[...] = lax.dot_general(wl_ref[...], h, _DN,
                                 preferred_element_type=jnp.float32)
    zo_ref[...] = lax.dot_general(wr_ref[...], h, _DN,
                                  preferred_element_type=jnp.float32) + b_ref[...]


def _head_body(agg_ref, z_ref, dinv_ref, a_ref, uv_ref, st_ref):
    pre = agg_ref[...] * dinv_ref[...] + z_ref[...]
    h = jnp.where(pre >= 0, pre, a_ref[...] * pre)
    st_ref[...] = lax.dot_general(uv_ref[...], h, _DN,
                                  preferred_element_type=jnp.float32)


def _dinv_body(cnt_ref, dinv_ref):
    s = jnp.sum(cnt_ref[...], axis=0, keepdims=True)
    dinv_ref[...] = 1.0 / jnp.maximum(s, 1.0)


def _final_body(part_ref, st_ref, dinv_ref, c_ref, out_ref):
    s = jnp.sum(part_ref[...], axis=0, keepdims=True)
    out_ref[...] = s * dinv_ref[...] + st_ref[1:2, :] + c_ref[...]


def _full(shape):
    return pl.BlockSpec(shape, lambda j: (0,) * len(shape))


def _tc_l1(x, wl, wr, b):
    return pl.pallas_call(
        _l1_body,
        grid=(NP // BLK,),
        in_specs=[pl.BlockSpec((BLK, D), lambda j: (j, 0)),
                  _full((D, D)), _full((D, D)), _full((D, 1))],
        out_specs=[pl.BlockSpec((D, BLK), lambda j: (0, j)),
                   pl.BlockSpec((D, BLK), lambda j: (0, j))],
        out_shape=[jax.ShapeDtypeStruct((D, NP), jnp.float32)] * 2,
    )(x, wl, wr, b)


def _tc_mid(agg, z, dinv, a, wl, wr, b):
    return pl.pallas_call(
        _mid_body,
        grid=(NP // BLK,),
        in_specs=[pl.BlockSpec((D, BLK), lambda j: (0, j)),
                  pl.BlockSpec((D, BLK), lambda j: (0, j)),
                  pl.BlockSpec((1, BLK), lambda j: (0, j)),
                  _full((1, 1)), _full((D, D)), _full((D, D)), _full((D, 1))],
        out_specs=[pl.BlockSpec((D, BLK), lambda j: (0, j)),
                   pl.BlockSpec((D, BLK), lambda j: (0, j))],
        out_shape=[jax.ShapeDtypeStruct((D, NP), jnp.float32)] * 2,
    )(agg, z, dinv, a, wl, wr, b)


def _tc_head(agg, z, dinv, a, uv):
    return pl.pallas_call(
        _head_body,
        grid=(NP // BLK,),
        in_specs=[pl.BlockSpec((D, BLK), lambda j: (0, j)),
                  pl.BlockSpec((D, BLK), lambda j: (0, j)),
                  pl.BlockSpec((1, BLK), lambda j: (0, j)),
                  _full((1, 1)), _full((2, D))],
        out_specs=pl.BlockSpec((2, BLK), lambda j: (0, j)),
        out_shape=jax.ShapeDtypeStruct((2, NP), jnp.float32),
    )(agg, z, dinv, a, uv)


def _tc_dinv(cnt):
    return pl.pallas_call(
        _dinv_body,
        grid=(NP // BLK,),
        in_specs=[pl.BlockSpec((NT, BLK), lambda j: (0, j))],
        out_specs=pl.BlockSpec((1, BLK), lambda j: (0, j)),
        out_shape=jax.ShapeDtypeStruct((1, NP), jnp.float32),
    )(cnt)


def _tc_final(part, st, dinv, c):
    return pl.pallas_call(
        _final_body,
        grid=(NP // BLK,),
        in_specs=[pl.BlockSpec((NT, BLK), lambda j: (0, j)),
                  pl.BlockSpec((2, BLK), lambda j: (0, j)),
                  pl.BlockSpec((1, BLK), lambda j: (0, j)),
                  _full((1, 1))],
        out_specs=pl.BlockSpec((1, BLK), lambda j: (0, j)),
        out_shape=jax.ShapeDtypeStruct((1, NP), jnp.float32),
    )(part, st, dinv, c)


def kernel(x, edge_index, W1l, b1, W1r, W2l, b2, W2r, W3l, b3, W3r, a, Wp, bp):
    src = edge_index[0]
    dst = edge_index[1]
    xp = jnp.pad(x, ((0, NP - N), (0, 0)))
    ones = jnp.ones((NP,), jnp.float32)
    a2 = jnp.reshape(a, (1, 1))
    b1c = jnp.reshape(b1, (D, 1))
    b2c = jnp.reshape(b2, (D, 1))
    uv = jnp.concatenate([Wp @ W3l, Wp @ W3r], axis=0)
    c = jnp.reshape(Wp @ b3 + bp, (1, 1))

    feat_segsum, scalar_segsum = _sc_kernels()
    cnt = scalar_segsum(ones, src, dst)
    dinv = _tc_dinv(cnt)
    y1, z1 = _tc_l1(xp, W1l, W1r, b1c)
    agg1 = feat_segsum(y1, src, dst)
    y2, z2 = _tc_mid(agg1, z1, dinv, a2, W2l, W2r, b2c)
    agg2 = feat_segsum(y2, src, dst)
    st = _tc_head(agg2, z2, dinv, a2, uv)
    spart = scalar_segsum(st[0], src, dst)
    out = _tc_final(spart, st, dinv, c)
    return out[0, :N]
